# baseline (speedup 1.0000x reference)
import functools

import jax
import jax.numpy as jnp
from jax.experimental import pallas as pl
from jax.experimental.pallas import tpu as pltpu


def _stage1_kernel(x_ref, w_ref, b_ref, s_ref, t_ref, o_ref, *, gb):
    for g in range(gb):
        groups = []
        for c in range(3):
            xcr = x_ref[g, c].astype(jnp.bfloat16).reshape(56, 4, 224)
            for ii in range(4):
                slab = xcr[:, ii, :]
                slab_t = slab.T
                groups.append(slab_t.reshape(56, 4, 56))
        s3all = jnp.concatenate(groups, axis=1)
        x_t = jnp.transpose(s3all, (0, 2, 1))
        pieces = [x_t[j:j + 54, i:i + 54, :]
                  for i in range(3) for j in range(3)]
        pat = jnp.concatenate(pieces, axis=-1)
        pat = pat.reshape(54 * 54, 432)
        acc = jnp.dot(pat, w_ref[...], preferred_element_type=jnp.float32)
        y = jnp.maximum(acc + b_ref[...], 0.0) * s_ref[...] + t_ref[...]
        y = y.reshape(54, 54, 128)
        rows = [jnp.max(y[2 * i:2 * i + 3], axis=0, keepdims=True)
                for i in range(26)]
        yh = jnp.concatenate(rows, axis=0)
        cols = [jnp.max(yh[:, 2 * j:2 * j + 3], axis=1, keepdims=True)
                for j in range(26)]
        out = jnp.concatenate(cols, axis=1)
        o_ref[g] = out.astype(o_ref.dtype)


def _stage2_kernel(x_ref, w_ref, b_ref, s_ref, t_ref, o_ref, *, gb):
    for g in range(gb):
        x = x_ref[g]
        xp = jnp.pad(x, ((2, 2), (2, 2), (0, 0)))
        pieces = [xp[j:j + 26, i:i + 26, :]
                  for i in range(5) for j in range(5)]
        pat = jnp.concatenate(pieces, axis=-1)
        pat = pat.reshape(26 * 26, 3200)
        acc = jnp.dot(pat, w_ref[...], preferred_element_type=jnp.float32)
        y = jnp.maximum(acc + b_ref[...], 0.0) * s_ref[...] + t_ref[...]
        y = y.reshape(26, 26, 256)
        rows = [jnp.max(y[2 * i:2 * i + 3], axis=0, keepdims=True)
                for i in range(12)]
        yh = jnp.concatenate(rows, axis=0)
        cols = [jnp.max(yh[:, 2 * j:2 * j + 3], axis=1, keepdims=True)
                for j in range(12)]
        out = jnp.concatenate(cols, axis=1)
        o_ref[g] = out.astype(o_ref.dtype)


def _conv3x3(x, w_ref, b_ref, bh):
    m = bh * 12 * 12
    k = 9 * x.shape[-1]
    xp = jnp.pad(x, ((0, 0), (1, 1), (1, 1), (0, 0)))
    pieces = [xp[:, j:j + 12, i:i + 12, :]
              for i in range(3) for j in range(3)]
    pat = jnp.concatenate(pieces, axis=-1).reshape(m, k)
    z = jnp.dot(pat, w_ref[...], preferred_element_type=jnp.float32)
    return jnp.maximum(z + b_ref[...], 0.0)


def _stage3_kernel(x_ref, w3_ref, b3_ref, w4_ref, b4_ref, w5_ref, b5_ref,
                   fw_ref, fb_ref, o_ref, *, bh):
    x = x_ref[...]
    h3 = _conv3x3(x, w3_ref, b3_ref, bh)
    h3 = h3.astype(jnp.bfloat16).reshape(bh, 12, 12, 384)
    h4 = _conv3x3(h3, w4_ref, b4_ref, bh)
    h4 = h4.astype(jnp.bfloat16).reshape(bh, 12, 12, 384)
    h5 = _conv3x3(h4, w5_ref, b5_ref, bh)
    h5 = h5.astype(jnp.bfloat16).astype(jnp.float32)
    g = jnp.mean(h5.reshape(bh, 144, 256), axis=1)
    z = jnp.dot(g.astype(jnp.bfloat16), fw_ref[...],
                preferred_element_type=jnp.float32) + fb_ref[...]
    o_ref[...] = jnp.maximum(z, 0.0).astype(o_ref.dtype)


def _fc3_kernel(x_ref, w_hbm, b_ref, o_ref, w_vmem, sems, *, kc, n):
    for kt in range(kc):
        pltpu.make_async_copy(
            w_hbm.at[pl.ds(kt * (4096 // kc), 4096 // kc), :],
            w_vmem.at[kt], sems.at[kt]).start()
    acc = b_ref[...]
    for kt in range(kc):
        pltpu.make_async_copy(
            w_hbm.at[pl.ds(kt * (4096 // kc), 4096 // kc), :],
            w_vmem.at[kt], sems.at[kt]).wait()
        xk = x_ref[:, kt * (4096 // kc):(kt + 1) * (4096 // kc)]
        acc = acc + jnp.dot(xk, w_vmem[kt],
                            preferred_element_type=jnp.float32)
    o_ref[...] = acc


def _fc_out(x, w, b):
    m = x.shape[0]
    k, n = w.shape
    kc = 4
    return pl.pallas_call(
        functools.partial(_fc3_kernel, kc=kc, n=n),
        grid=(1,),
        in_specs=[
            pl.BlockSpec((m, k), lambda i: (0, 0)),
            pl.BlockSpec(memory_space=pl.ANY),
            pl.BlockSpec((1, n), lambda i: (0, 0)),
        ],
        out_specs=pl.BlockSpec((m, n), lambda i: (0, 0)),
        out_shape=jax.ShapeDtypeStruct((m, n), jnp.float32),
        scratch_shapes=[
            pltpu.VMEM((kc, k // kc, n), jnp.bfloat16),
            pltpu.SemaphoreType.DMA((kc,)),
        ],
        compiler_params=pltpu.CompilerParams(
            dimension_semantics=("arbitrary",),
            vmem_limit_bytes=48 * 1024 * 1024),
    )(x, w, b.reshape(1, n))


def _fc_kernel(x_ref, w_ref, b_ref, o_ref, *, relu):
    z = jnp.dot(x_ref[...], w_ref[...], preferred_element_type=jnp.float32)
    z = z + b_ref[...]
    if relu:
        z = jnp.maximum(z, 0.0)
    o_ref[...] = z.astype(o_ref.dtype)


def _fc(x, w, b, relu, out_dtype, tn):
    m = x.shape[0]
    k, n = w.shape
    nt = pl.cdiv(n, tn)
    grid = (2, nt // 2)
    return pl.pallas_call(
        functools.partial(_fc_kernel, relu=relu),
        grid=grid,
        in_specs=[
            pl.BlockSpec((m, k), lambda i, j: (0, 0)),
            pl.BlockSpec((k, tn), lambda i, j: (0, i * (nt // 2) + j)),
            pl.BlockSpec((1, tn), lambda i, j: (0, i * (nt // 2) + j)),
        ],
        out_specs=pl.BlockSpec((m, tn), lambda i, j: (0, i * (nt // 2) + j)),
        out_shape=jax.ShapeDtypeStruct((m, n), out_dtype),
        compiler_params=pltpu.CompilerParams(
            dimension_semantics=("parallel", "arbitrary"),
            vmem_limit_bytes=48 * 1024 * 1024),
    )(x, w, b.reshape(1, n))


def kernel(x_nchw, w1, b1, w2, b2, w3, b3, w4, b4, w5, b5,
           fc1_w, fc1_b, fc2_w, fc2_b, out_w, out_b,
           bn1_scale, bn1_shift, bn2_scale, bn2_shift):
    B = x_nchw.shape[0]
    f32 = jnp.float32

    w1r = jnp.pad(w1.reshape(11, 11, 3, 96), ((0, 1), (0, 1), (0, 0), (0, 32)))
    w1p = (w1r.reshape(3, 4, 3, 4, 3, 128)
           .transpose(0, 2, 4, 1, 3, 5)
           .reshape(432, 128))
    b1p = jnp.pad(b1.astype(f32), (0, 32)).reshape(1, 128)
    s1p = jnp.pad(bn1_scale.astype(f32), (0, 32)).reshape(1, 128)
    t1p = jnp.pad(bn1_shift.astype(f32), (0, 32)).reshape(1, 128)

    gb = 2
    h1 = pl.pallas_call(
        functools.partial(_stage1_kernel, gb=gb),
        grid=(B // gb,),
        in_specs=[
            pl.BlockSpec((gb, 3, 224, 224), lambda i: (i, 0, 0, 0)),
            pl.BlockSpec((432, 128), lambda i: (0, 0)),
            pl.BlockSpec((1, 128), lambda i: (0, 0)),
            pl.BlockSpec((1, 128), lambda i: (0, 0)),
            pl.BlockSpec((1, 128), lambda i: (0, 0)),
        ],
        out_specs=pl.BlockSpec((gb, 26, 26, 128), lambda i: (i, 0, 0, 0)),
        out_shape=jax.ShapeDtypeStruct((B, 26, 26, 128), jnp.bfloat16),
        compiler_params=pltpu.CompilerParams(
            dimension_semantics=("arbitrary",),
            vmem_limit_bytes=40 * 1024 * 1024),
    )(x_nchw, w1p, b1p, s1p, t1p)

    w2p = jnp.pad(w2.reshape(5, 5, 96, 256),
                  ((0, 0), (0, 0), (0, 32), (0, 0))).reshape(3200, 256)
    h2 = pl.pallas_call(
        functools.partial(_stage2_kernel, gb=gb),
        grid=(B // gb,),
        in_specs=[
            pl.BlockSpec((gb, 26, 26, 128), lambda i: (i, 0, 0, 0)),
            pl.BlockSpec((3200, 256), lambda i: (0, 0)),
            pl.BlockSpec((1, 256), lambda i: (0, 0)),
            pl.BlockSpec((1, 256), lambda i: (0, 0)),
            pl.BlockSpec((1, 256), lambda i: (0, 0)),
        ],
        out_specs=pl.BlockSpec((gb, 12, 12, 256), lambda i: (i, 0, 0, 0)),
        out_shape=jax.ShapeDtypeStruct((B, 12, 12, 256), jnp.bfloat16),
        compiler_params=pltpu.CompilerParams(
            dimension_semantics=("arbitrary",),
            vmem_limit_bytes=40 * 1024 * 1024),
    )(h1, w2p, b2.astype(f32).reshape(1, 256),
      bn2_scale.astype(f32).reshape(1, 256),
      bn2_shift.astype(f32).reshape(1, 256))

    bh = B // 2
    h6 = pl.pallas_call(
        functools.partial(_stage3_kernel, bh=bh),
        grid=(2,),
        in_specs=[
            pl.BlockSpec((bh, 12, 12, 256), lambda i: (i, 0, 0, 0)),
            pl.BlockSpec((2304, 384), lambda i: (0, 0)),
            pl.BlockSpec((1, 384), lambda i: (0, 0)),
            pl.BlockSpec((3456, 384), lambda i: (0, 0)),
            pl.BlockSpec((1, 384), lambda i: (0, 0)),
            pl.BlockSpec((3456, 256), lambda i: (0, 0)),
            pl.BlockSpec((1, 256), lambda i: (0, 0)),
            pl.BlockSpec((256, 4096), lambda i: (0, 0)),
            pl.BlockSpec((1, 4096), lambda i: (0, 0)),
        ],
        out_specs=pl.BlockSpec((bh, 4096), lambda i: (i, 0)),
        out_shape=jax.ShapeDtypeStruct((B, 4096), jnp.bfloat16),
        compiler_params=pltpu.CompilerParams(
            dimension_semantics=("parallel",),
            vmem_limit_bytes=56 * 1024 * 1024),
    )(h2, w3, b3.astype(f32).reshape(1, 384),
      w4, b4.astype(f32).reshape(1, 384),
      w5, b5.astype(f32).reshape(1, 256),
      fc1_w, fc1_b.astype(f32).reshape(1, 4096))

    h7 = _fc(h6, fc2_w, fc2_b.astype(f32), relu=True,
             out_dtype=jnp.bfloat16, tn=1024)
    out = _fc_out(h7, out_w, out_b.astype(f32))
    return out


# speedup vs baseline: 1.0811x; 1.0811x over previous
import functools

import jax
import jax.numpy as jnp
from jax.experimental import pallas as pl
from jax.experimental.pallas import tpu as pltpu


def _stage1_kernel(x_ref, w_ref, b_ref, s_ref, t_ref, o_ref, *, gb):
    for g in range(gb):
        groups = []
        for c in range(3):
            xcr = x_ref[g, c].astype(jnp.bfloat16).reshape(56, 4, 224)
            for ii in range(4):
                slab = xcr[:, ii, :]
                slab_t = slab.T
                groups.append(slab_t.reshape(56, 4, 56))
        s3all = jnp.concatenate(groups, axis=1)
        x_t = jnp.transpose(s3all, (0, 2, 1))
        pieces = [x_t[j:j + 54, i:i + 54, :]
                  for i in range(3) for j in range(3)]
        pat = jnp.concatenate(pieces, axis=-1)
        pat = pat.reshape(54 * 54, 432)
        acc = jnp.dot(pat, w_ref[...], preferred_element_type=jnp.float32)
        y = jnp.maximum(acc + b_ref[...], 0.0) * s_ref[...] + t_ref[...]
        y = y.reshape(54, 54, 128)
        rows = [jnp.max(y[2 * i:2 * i + 3], axis=0, keepdims=True)
                for i in range(26)]
        yh = jnp.concatenate(rows, axis=0)
        cols = [jnp.max(yh[:, 2 * j:2 * j + 3], axis=1, keepdims=True)
                for j in range(26)]
        out = jnp.concatenate(cols, axis=1)
        o_ref[g] = out.astype(o_ref.dtype)


def _stage2_kernel(x_ref, w_ref, b_ref, s_ref, t_ref, o_ref, *, gb):
    for g in range(gb):
        x = x_ref[g]
        xp = jnp.pad(x, ((2, 2), (2, 2), (0, 0)))
        pieces = [xp[j:j + 26, i:i + 26, :]
                  for i in range(5) for j in range(5)]
        pat = jnp.concatenate(pieces, axis=-1)
        pat = pat.reshape(26 * 26, 3200)
        acc = jnp.dot(pat, w_ref[...], preferred_element_type=jnp.float32)
        y = jnp.maximum(acc + b_ref[...], 0.0) * s_ref[...] + t_ref[...]
        y = y.reshape(26, 26, 256)
        rows = [jnp.max(y[2 * i:2 * i + 3], axis=0, keepdims=True)
                for i in range(12)]
        yh = jnp.concatenate(rows, axis=0)
        cols = [jnp.max(yh[:, 2 * j:2 * j + 3], axis=1, keepdims=True)
                for j in range(12)]
        out = jnp.concatenate(cols, axis=1)
        o_ref[g] = out.astype(o_ref.dtype)


def _conv3x3(x, w_ref, b_ref, bh):
    m = bh * 12 * 12
    k = 9 * x.shape[-1]
    xp = jnp.pad(x, ((0, 0), (1, 1), (1, 1), (0, 0)))
    pieces = [xp[:, j:j + 12, i:i + 12, :]
              for i in range(3) for j in range(3)]
    pat = jnp.concatenate(pieces, axis=-1).reshape(m, k)
    z = jnp.dot(pat, w_ref[...], preferred_element_type=jnp.float32)
    return jnp.maximum(z + b_ref[...], 0.0)


def _stage3_kernel(x_ref, w3_ref, b3_ref, w4_ref, b4_ref, w5_ref, b5_ref,
                   fw_ref, fb_ref, o_ref, *, bh):
    x = x_ref[...]
    h3 = _conv3x3(x, w3_ref, b3_ref, bh)
    h3 = h3.astype(jnp.bfloat16).reshape(bh, 12, 12, 384)
    h4 = _conv3x3(h3, w4_ref, b4_ref, bh)
    h4 = h4.astype(jnp.bfloat16).reshape(bh, 12, 12, 384)
    h5 = _conv3x3(h4, w5_ref, b5_ref, bh)
    h5 = h5.astype(jnp.bfloat16).astype(jnp.float32)
    g = jnp.mean(h5.reshape(bh, 144, 256), axis=1)
    z = jnp.dot(g.astype(jnp.bfloat16), fw_ref[...],
                preferred_element_type=jnp.float32) + fb_ref[...]
    o_ref[...] = jnp.maximum(z, 0.0).astype(o_ref.dtype)


def _fc3_kernel(x_ref, wt_hbm, b_ref, o_ref, w_vmem, sems, *, kc, kw):
    for kt in range(kc):
        pltpu.make_async_copy(
            wt_hbm.at[:, pl.ds(kt * kw, kw)],
            w_vmem.at[kt], sems.at[kt]).start()
    acc = b_ref[...]
    for kt in range(kc):
        pltpu.make_async_copy(
            wt_hbm.at[:, pl.ds(kt * kw, kw)],
            w_vmem.at[kt], sems.at[kt]).wait()
        xk = x_ref[:, kt * kw:(kt + 1) * kw]
        acc = acc + jax.lax.dot_general(
            xk, w_vmem[kt], (((1,), (1,)), ((), ())),
            preferred_element_type=jnp.float32)
    o_ref[...] = acc


def _fc_out(x, w_t, b):
    m = x.shape[0]
    n, k = w_t.shape
    kc = 4
    kw = k // kc
    return pl.pallas_call(
        functools.partial(_fc3_kernel, kc=kc, kw=kw),
        grid=(1,),
        in_specs=[
            pl.BlockSpec((m, k), lambda i: (0, 0)),
            pl.BlockSpec(memory_space=pl.ANY),
            pl.BlockSpec((1, n), lambda i: (0, 0)),
        ],
        out_specs=pl.BlockSpec((m, n), lambda i: (0, 0)),
        out_shape=jax.ShapeDtypeStruct((m, n), jnp.float32),
        scratch_shapes=[
            pltpu.VMEM((kc, n, kw), jnp.bfloat16),
            pltpu.SemaphoreType.DMA((kc,)),
        ],
        compiler_params=pltpu.CompilerParams(
            dimension_semantics=("arbitrary",),
            vmem_limit_bytes=48 * 1024 * 1024),
    )(x, w_t, b.reshape(1, n))


def _fc_kernel(x_ref, w_ref, b_ref, o_ref, *, relu):
    z = jnp.dot(x_ref[...], w_ref[...], preferred_element_type=jnp.float32)
    z = z + b_ref[...]
    if relu:
        z = jnp.maximum(z, 0.0)
    o_ref[...] = z.astype(o_ref.dtype)


def _fc(x, w, b, relu, out_dtype, tn):
    m = x.shape[0]
    k, n = w.shape
    nt = pl.cdiv(n, tn)
    grid = (2, nt // 2)
    return pl.pallas_call(
        functools.partial(_fc_kernel, relu=relu),
        grid=grid,
        in_specs=[
            pl.BlockSpec((m, k), lambda i, j: (0, 0)),
            pl.BlockSpec((k, tn), lambda i, j: (0, i * (nt // 2) + j)),
            pl.BlockSpec((1, tn), lambda i, j: (0, i * (nt // 2) + j)),
        ],
        out_specs=pl.BlockSpec((m, tn), lambda i, j: (0, i * (nt // 2) + j)),
        out_shape=jax.ShapeDtypeStruct((m, n), out_dtype),
        compiler_params=pltpu.CompilerParams(
            dimension_semantics=("parallel", "arbitrary"),
            vmem_limit_bytes=48 * 1024 * 1024),
    )(x, w, b.reshape(1, n))


def kernel(x_nchw, w1, b1, w2, b2, w3, b3, w4, b4, w5, b5,
           fc1_w, fc1_b, fc2_w, fc2_b, out_w, out_b,
           bn1_scale, bn1_shift, bn2_scale, bn2_shift):
    B = x_nchw.shape[0]
    f32 = jnp.float32

    w1r = jnp.pad(w1.reshape(11, 11, 3, 96), ((0, 1), (0, 1), (0, 0), (0, 32)))
    w1p = (w1r.reshape(3, 4, 3, 4, 3, 128)
           .transpose(0, 2, 4, 1, 3, 5)
           .reshape(432, 128))
    b1p = jnp.pad(b1.astype(f32), (0, 32)).reshape(1, 128)
    s1p = jnp.pad(bn1_scale.astype(f32), (0, 32)).reshape(1, 128)
    t1p = jnp.pad(bn1_shift.astype(f32), (0, 32)).reshape(1, 128)

    gb = 2
    h1 = pl.pallas_call(
        functools.partial(_stage1_kernel, gb=gb),
        grid=(B // gb,),
        in_specs=[
            pl.BlockSpec((gb, 3, 224, 224), lambda i: (i, 0, 0, 0)),
            pl.BlockSpec((432, 128), lambda i: (0, 0)),
            pl.BlockSpec((1, 128), lambda i: (0, 0)),
            pl.BlockSpec((1, 128), lambda i: (0, 0)),
            pl.BlockSpec((1, 128), lambda i: (0, 0)),
        ],
        out_specs=pl.BlockSpec((gb, 26, 26, 128), lambda i: (i, 0, 0, 0)),
        out_shape=jax.ShapeDtypeStruct((B, 26, 26, 128), jnp.bfloat16),
        compiler_params=pltpu.CompilerParams(
            dimension_semantics=("arbitrary",),
            vmem_limit_bytes=40 * 1024 * 1024),
    )(x_nchw, w1p, b1p, s1p, t1p)

    w2p = jnp.pad(w2.reshape(5, 5, 96, 256),
                  ((0, 0), (0, 0), (0, 32), (0, 0))).reshape(3200, 256)
    h2 = pl.pallas_call(
        functools.partial(_stage2_kernel, gb=gb),
        grid=(B // gb,),
        in_specs=[
            pl.BlockSpec((gb, 26, 26, 128), lambda i: (i, 0, 0, 0)),
            pl.BlockSpec((3200, 256), lambda i: (0, 0)),
            pl.BlockSpec((1, 256), lambda i: (0, 0)),
            pl.BlockSpec((1, 256), lambda i: (0, 0)),
            pl.BlockSpec((1, 256), lambda i: (0, 0)),
        ],
        out_specs=pl.BlockSpec((gb, 12, 12, 256), lambda i: (i, 0, 0, 0)),
        out_shape=jax.ShapeDtypeStruct((B, 12, 12, 256), jnp.bfloat16),
        compiler_params=pltpu.CompilerParams(
            dimension_semantics=("arbitrary",),
            vmem_limit_bytes=40 * 1024 * 1024),
    )(h1, w2p, b2.astype(f32).reshape(1, 256),
      bn2_scale.astype(f32).reshape(1, 256),
      bn2_shift.astype(f32).reshape(1, 256))

    bh = B // 2
    h6 = pl.pallas_call(
        functools.partial(_stage3_kernel, bh=bh),
        grid=(2,),
        in_specs=[
            pl.BlockSpec((bh, 12, 12, 256), lambda i: (i, 0, 0, 0)),
            pl.BlockSpec((2304, 384), lambda i: (0, 0)),
            pl.BlockSpec((1, 384), lambda i: (0, 0)),
            pl.BlockSpec((3456, 384), lambda i: (0, 0)),
            pl.BlockSpec((1, 384), lambda i: (0, 0)),
            pl.BlockSpec((3456, 256), lambda i: (0, 0)),
            pl.BlockSpec((1, 256), lambda i: (0, 0)),
            pl.BlockSpec((256, 4096), lambda i: (0, 0)),
            pl.BlockSpec((1, 4096), lambda i: (0, 0)),
        ],
        out_specs=pl.BlockSpec((bh, 4096), lambda i: (i, 0)),
        out_shape=jax.ShapeDtypeStruct((B, 4096), jnp.bfloat16),
        compiler_params=pltpu.CompilerParams(
            dimension_semantics=("parallel",),
            vmem_limit_bytes=56 * 1024 * 1024),
    )(h2, w3, b3.astype(f32).reshape(1, 384),
      w4, b4.astype(f32).reshape(1, 384),
      w5, b5.astype(f32).reshape(1, 256),
      fc1_w, fc1_b.astype(f32).reshape(1, 4096))

    h7 = _fc(h6, fc2_w, fc2_b.astype(f32), relu=True,
             out_dtype=jnp.bfloat16, tn=1024)
    out = _fc_out(h7, out_w.T, out_b.astype(f32))
    return out
